# Initial kernel scaffold; baseline (speedup 1.0000x reference)
import jax
import jax.numpy as jnp
from jax.experimental import pallas as pl


def kernel(h, gene_pos, conv_w, conv_b):
    raise NotImplementedError("write your pallas kernel here")


# speedup vs baseline: 2.5301x; 2.5301x over previous
import functools

import jax
import jax.numpy as jnp
from jax import lax
from jax.experimental import pallas as pl
from jax.experimental.pallas import tpu as pltpu
from jax.experimental.pallas import tpu_sc as plsc

B, S, D = 16, 2048, 512
FULL = 60000
HALF = FULL // 2
LANES = 16


def _pool_body(h_ref, w_ref, b_ref, out_ref):
    hb = h_ref[...]
    w = w_ref[...][None, None, :]
    out_ref[...] = jnp.sum(hb * w, axis=-1) + b_ref[0]


def _pooled_tc(h, conv_w, conv_b, blk=16):
    n = B * S // 128
    h3 = h.reshape(n, 128, D)
    pooled = pl.pallas_call(
        _pool_body,
        grid=(n // blk,),
        in_specs=[
            pl.BlockSpec((blk, 128, D), lambda i: (i, 0, 0)),
            pl.BlockSpec((D,), lambda i: (0,)),
            pl.BlockSpec((1,), lambda i: (0,)),
        ],
        out_specs=pl.BlockSpec((blk, 128), lambda i: (i, 0)),
        out_shape=jax.ShapeDtypeStruct((n, 128), jnp.float32),
    )(h3, conv_w, conv_b)
    return pooled.reshape(B, S)


_MESH = plsc.VectorSubcoreMesh(core_axis_name="c", subcore_axis_name="s")


@functools.partial(
    pl.kernel,
    mesh=_MESH,
    compiler_params=pltpu.CompilerParams(needs_layout_passes=False),
    out_type=jax.ShapeDtypeStruct((B * FULL,), jnp.float32),
    scratch_types=[
        pltpu.VMEM((S,), jnp.int32),
        pltpu.VMEM((S,), jnp.float32),
        pltpu.VMEM((HALF,), jnp.float32),
    ],
)
def _scatter_sc(idx_hbm, val_hbm, out_hbm, idx_v, val_v, row_v):
    wid = lax.axis_index("c") * 16 + lax.axis_index("s")
    b = wid // 2
    lo = (wid % 2) * HALF

    pltpu.sync_copy(idx_hbm.at[pl.ds(b * S, S)], idx_v)
    pltpu.sync_copy(val_hbm.at[pl.ds(b * S, S)], val_v)

    zeros = jnp.zeros((LANES,), jnp.float32)

    def zero_body(j, _):
        row_v[pl.ds(j * LANES, LANES)] = zeros
        return 0

    lax.fori_loop(0, HALF // LANES, zero_body, 0)

    def scat_body(i, _):
        idx = idx_v[pl.ds(i * LANES, LANES)]
        val = val_v[pl.ds(i * LANES, LANES)]
        local = idx - lo
        mask = (local >= 0) & (local < HALF)
        plsc.store_scatter(row_v, [local], val, mask=mask)
        return 0

    lax.fori_loop(0, S // LANES, scat_body, 0)

    pltpu.sync_copy(row_v, out_hbm.at[pl.ds(b * FULL + lo, HALF)])


def kernel(h, gene_pos, conv_w, conv_b):
    pooled = _pooled_tc(h, conv_w, conv_b)
    out = _scatter_sc(gene_pos.reshape(B * S), pooled.reshape(B * S))
    return out.reshape(B, FULL)


# speedup vs baseline: 3.0242x; 1.1953x over previous
import functools

import jax
import jax.numpy as jnp
from jax import lax
from jax.experimental import pallas as pl
from jax.experimental.pallas import tpu as pltpu
from jax.experimental.pallas import tpu_sc as plsc

B, S, D = 16, 2048, 512
FULL = 60000
HALF = FULL // 2
LANES = 16


def _pool_body(h_ref, w_ref, b_ref, out_ref):
    hb = h_ref[...]
    w = w_ref[...][None, None, :]
    out_ref[...] = jnp.sum(hb * w, axis=-1) + b_ref[0]


def _pooled_tc(h, conv_w, conv_b, blk=32):
    n = B * S // 128
    h3 = h.reshape(n, 128, D)
    pooled = pl.pallas_call(
        _pool_body,
        grid=(n // blk,),
        in_specs=[
            pl.BlockSpec((blk, 128, D), lambda i: (i, 0, 0)),
            pl.BlockSpec((D,), lambda i: (0,)),
            pl.BlockSpec((1,), lambda i: (0,)),
        ],
        out_specs=pl.BlockSpec((blk, 128), lambda i: (i, 0)),
        out_shape=jax.ShapeDtypeStruct((n, 128), jnp.float32),
    )(h3, conv_w, conv_b)
    return pooled.reshape(B, S)


_MESH = plsc.VectorSubcoreMesh(core_axis_name="c", subcore_axis_name="s")


@functools.partial(
    pl.kernel,
    mesh=_MESH,
    compiler_params=pltpu.CompilerParams(needs_layout_passes=False),
    out_type=jax.ShapeDtypeStruct((B * FULL,), jnp.float32),
    scratch_types=[
        pltpu.VMEM((S,), jnp.int32),
        pltpu.VMEM((S,), jnp.float32),
        pltpu.VMEM((HALF,), jnp.float32),
        pltpu.SemaphoreType.DMA,
        pltpu.SemaphoreType.DMA,
    ],
)
def _scatter_sc(idx_hbm, val_hbm, out_hbm, idx_v, val_v, row_v, sem_i, sem_v):
    wid = lax.axis_index("c") * 16 + lax.axis_index("s")
    b = wid // 2
    lo = (wid % 2) * HALF

    cp_i = pltpu.async_copy(idx_hbm.at[pl.ds(b * S, S)], idx_v, sem_i)
    cp_v = pltpu.async_copy(val_hbm.at[pl.ds(b * S, S)], val_v, sem_v)

    zeros = jnp.zeros((LANES,), jnp.float32)

    def zero_body(j, _):
        base = j * (15 * LANES)
        for u in range(15):
            row_v[pl.ds(base + u * LANES, LANES)] = zeros
        return 0

    lax.fori_loop(0, HALF // (15 * LANES), zero_body, 0)

    cp_i.wait()
    cp_v.wait()

    def scat_body(i, _):
        base = i * (4 * LANES)
        for u in range(4):
            idx = idx_v[pl.ds(base + u * LANES, LANES)]
            val = val_v[pl.ds(base + u * LANES, LANES)]
            local = idx - lo
            mask = (local >= 0) & (local < HALF)
            plsc.store_scatter(row_v, [local], val, mask=mask)
        return 0

    lax.fori_loop(0, S // (4 * LANES), scat_body, 0)

    pltpu.sync_copy(row_v, out_hbm.at[pl.ds(b * FULL + lo, HALF)])


def kernel(h, gene_pos, conv_w, conv_b):
    pooled = _pooled_tc(h, conv_w, conv_b)
    out = _scatter_sc(gene_pos.reshape(B * S), pooled.reshape(B * S))
    return out.reshape(B, FULL)
